# baseline (speedup 1.0000x reference)
import jax
import jax.numpy as jnp
from jax.experimental import pallas as pl
from jax.experimental.pallas import tpu as pltpu

B, S, K, D = 1024, 50, 20, 128
BB = 256


def _entity_scan_kernel(x_ref, mb_ref, keys_ref, u_ref, v_ref, w_ref,
                        out_ref, xw_s, mb_s):
    keys_f = keys_ref[...]
    u = u_ref[...]
    v = v_ref[...]
    w = w_ref[...]

    kv = jnp.dot(keys_f.reshape(K * BB, D), v,
                 preferred_element_type=jnp.float32
                 ).reshape(K, BB, D).astype(jnp.bfloat16)
    keys = keys_f.astype(jnp.bfloat16)
    x_all = x_ref[...]
    xw_s[...] = jnp.dot(x_all.reshape(S * BB, D), w,
                        preferred_element_type=jnp.float32).reshape(S, BB, D)
    mb_s[...] = jnp.broadcast_to(mb_ref[...][:, :, None], (S, BB, D))

    ones = jnp.ones((D, D), dtype=jnp.float32)

    def step(t, h):
        x_t = x_ref[pl.ds(t, 1)]
        hkx = (h + keys) * x_t
        gsum = jnp.dot(hkx.reshape(K * BB, D), ones,
                       preferred_element_type=jnp.float32).reshape(K, BB, D)
        g = 0.5 * jnp.tanh(gsum + mb_s[pl.ds(t, 1)]) + 0.5
        hu = jnp.dot(h.reshape(K * BB, D), u,
                     preferred_element_type=jnp.float32).reshape(K, BB, D)
        h_tilda = jax.nn.relu(hu + kv + xw_s[pl.ds(t, 1)])
        upd = h + g * h_tilda
        ss = jnp.dot((upd * upd).reshape(K * BB, D), ones,
                     preferred_element_type=jnp.float32).reshape(K, BB, D)
        return upd * jax.lax.rsqrt(jnp.maximum(ss, 1e-12))

    h0 = jnp.zeros((K, BB, D), dtype=jnp.float32)
    out_ref[...] = jax.lax.fori_loop(0, S, step, h0)


@jax.jit
def kernel(encoded_sents, mask, keys, U, V, W):
    nb = B // BB
    x_t = jnp.swapaxes(encoded_sents, 0, 1) * 0.5
    mask_bias = (mask.T.astype(jnp.float32) - 1.0) * 1e30
    keys_t = jnp.swapaxes(keys, 0, 1)
    W = W * 2.0
    out = pl.pallas_call(
        _entity_scan_kernel,
        grid=(nb,),
        in_specs=[
            pl.BlockSpec((S, BB, D), lambda b: (0, b, 0)),
            pl.BlockSpec((S, BB), lambda b: (0, b)),
            pl.BlockSpec((K, BB, D), lambda b: (0, b, 0)),
            pl.BlockSpec((D, D), lambda b: (0, 0)),
            pl.BlockSpec((D, D), lambda b: (0, 0)),
            pl.BlockSpec((D, D), lambda b: (0, 0)),
        ],
        out_specs=pl.BlockSpec((K, BB, D), lambda b: (0, b, 0)),
        out_shape=jax.ShapeDtypeStruct((K, B, D), jnp.float32),
        scratch_shapes=[
            pltpu.VMEM((S, BB, D), jnp.float32),
            pltpu.VMEM((S, BB, D), jnp.float32),
        ],
    )(x_t, mask_bias, keys_t, U, V, W)
    return jnp.swapaxes(out, 0, 1)


# speedup vs baseline: 1.0481x; 1.0481x over previous
import jax
import jax.numpy as jnp
from jax.experimental import pallas as pl
from jax.experimental.pallas import tpu as pltpu

B, S, K, D = 1024, 50, 20, 128
BB = 256


def _entity_scan_kernel(x_ref, mb_ref, keys_ref, u_ref, v_ref, w_ref,
                        out_ref, xw_s, mb_s):
    keys = keys_ref[...]
    u = u_ref[...]
    v = v_ref[...]
    w = w_ref[...]

    kv = jnp.dot(keys.reshape(K * BB, D), v,
                 preferred_element_type=jnp.float32).reshape(K, BB, D)
    x_all = x_ref[...]
    xw_s[...] = jnp.dot(x_all.reshape(S * BB, D), w,
                        preferred_element_type=jnp.float32).reshape(S, BB, D)
    mb_s[...] = jnp.broadcast_to(mb_ref[...][:, :, None], (S, BB, D))

    ones = jnp.ones((D, D), dtype=jnp.float32)

    def step(t, h):
        x_t = x_ref[pl.ds(t, 1)]
        hkx = (h + keys) * x_t
        gsum = jnp.dot(hkx.reshape(K * BB, D), ones,
                       preferred_element_type=jnp.float32).reshape(K, BB, D)
        g = 0.5 * jnp.tanh(gsum + mb_s[pl.ds(t, 1)]) + 0.5
        hu = jnp.dot(h.reshape(K * BB, D), u,
                     preferred_element_type=jnp.float32).reshape(K, BB, D)
        h_tilda = jax.nn.relu(hu + kv + xw_s[pl.ds(t, 1)])
        upd = h + g * h_tilda
        ss = jnp.dot((upd * upd).reshape(K * BB, D), ones,
                     preferred_element_type=jnp.float32).reshape(K, BB, D)
        return upd * jax.lax.rsqrt(jnp.maximum(ss, 1e-12))

    h0 = jnp.zeros((K, BB, D), dtype=jnp.float32)
    out_ref[...] = jax.lax.fori_loop(0, S, step, h0)


@jax.jit
def kernel(encoded_sents, mask, keys, U, V, W):
    nb = B // BB
    x_t = jnp.swapaxes(encoded_sents, 0, 1) * 0.5
    mask_bias = (mask.T.astype(jnp.float32) - 1.0) * 1e30
    keys_t = jnp.swapaxes(keys, 0, 1)
    W = W * 2.0
    out = pl.pallas_call(
        _entity_scan_kernel,
        grid=(nb,),
        in_specs=[
            pl.BlockSpec((S, BB, D), lambda b: (0, b, 0)),
            pl.BlockSpec((S, BB), lambda b: (0, b)),
            pl.BlockSpec((K, BB, D), lambda b: (0, b, 0)),
            pl.BlockSpec((D, D), lambda b: (0, 0)),
            pl.BlockSpec((D, D), lambda b: (0, 0)),
            pl.BlockSpec((D, D), lambda b: (0, 0)),
        ],
        out_specs=pl.BlockSpec((K, BB, D), lambda b: (0, b, 0)),
        out_shape=jax.ShapeDtypeStruct((K, B, D), jnp.float32),
        scratch_shapes=[
            pltpu.VMEM((S, BB, D), jnp.float32),
            pltpu.VMEM((S, BB, D), jnp.float32),
        ],
    )(x_t, mask_bias, keys_t, U, V, W)
    return jnp.swapaxes(out, 0, 1)
